# baseline (speedup 1.0000x reference)
import functools

import jax
import jax.numpy as jnp
from jax import lax
from jax.experimental import pallas as pl
from jax.experimental.pallas import tpu as pltpu
from jax.experimental.pallas import tpu_sc as plsc

_CHUNK = 128


def _sc_segment_sum(x, ei, zblk, zcnt):
    n, d = x.shape
    nc2, e = ei.shape
    info = plsc.get_sparse_core_info()
    nc, ns = info.num_cores, info.num_subcores
    nw = nc * ns
    nch_all = e // _CHUNK
    q, r = divmod(nch_all, nw)
    half = _CHUNK // 2
    rps = zblk.shape[0]
    np_ = rps * ns
    assert nch_all * _CHUNK == e and nc2 == 2 and rps % 16 == 0 and np_ >= n
    assert 0 < n - (ns - 1) * rps <= rps and (n - (ns - 1) * rps) % 16 == 0
    assert n % 16 == 0 and q >= 4

    mesh = plsc.VectorSubcoreMesh(core_axis_name="c", subcore_axis_name="s")

    @functools.partial(
        pl.kernel,
        mesh=mesh,
        out_type=[
            jax.ShapeDtypeStruct((nc * n, d), jnp.float32),
            jax.ShapeDtypeStruct((nc * np_,), jnp.float32),
        ],
        scratch_types=[
            pltpu.VMEM_SHARED((np_, d), jnp.float32),
            pltpu.VMEM_SHARED((np_,), jnp.float32),
        ]
        + [pltpu.VMEM((2, _CHUNK), jnp.int32)] * 2
        + [pltpu.VMEM((half,), jnp.int32)] * 8
        + [pltpu.VMEM((half, d), jnp.float32)] * 4
        + [pltpu.VMEM((half,), jnp.float32)]
        + [pltpu.SemaphoreType.DMA] * 10,
    )
    def seg_sum(x_hbm, ei_hbm, z_hbm, zc_hbm, out_hbm, outc_hbm,
                acc, cacc, eib0, eib1, si0, si1, si2, si3, di0, di1, di2, di3,
                rows0, rows1, rows2, rows3, ones_v,
                sem_e0, sem_e1, sg0, sg1, sg2, sg3, ss0, ss1, ss2, ss3):
        eib = (eib0, eib1)
        sidx = (si0, si1, si2, si3)
        didx = (di0, di1, di2, di3)
        rows = (rows0, rows1, rows2, rows3)
        sem_e = (sem_e0, sem_e1)
        sem_g = (sg0, sg1, sg2, sg3)
        sem_s = (ss0, ss1, ss2, ss3)
        cid = lax.axis_index("c")
        sid = lax.axis_index("s")
        pltpu.sync_copy(z_hbm, acc.at[pl.ds(sid * rps, rps)])
        pltpu.sync_copy(zc_hbm, cacc.at[pl.ds(sid * rps, rps)])
        for j in range(half // 16):
            ones_v[pl.ds(16 * j, 16)] = jnp.ones((16,), jnp.float32)
        wid = cid * ns + sid
        nch = q + jnp.where(wid < r, 1, 0)
        b0 = wid * q + jnp.minimum(wid, r)
        plsc.subcore_barrier()

        def ei_desc(c, me):
            return pltpu.make_async_copy(
                ei_hbm.at[:, pl.ds(c * _CHUNK, _CHUNK)], eib[me], sem_e[me])

        def extract(me, m):
            for hh in range(2):
                for j in range(half // 16):
                    sl = pl.ds(hh * half + 16 * j, 16)
                    so = pl.ds(16 * j, 16)
                    sidx[m + hh][so] = eib[me][0, sl]
                    didx[m + hh][so] = eib[me][1, sl]

        def gather_desc(m):
            return pltpu.make_async_copy(x_hbm.at[sidx[m]], rows[m], sem_g[m])

        def row_scat_desc(m):
            return pltpu.make_async_copy(rows[m], acc.at[didx[m]], sem_s[m])

        def cnt_scat_desc(m):
            return pltpu.make_async_copy(ones_v, cacc.at[didx[m]], sem_s[m])

        def fire_scats(m):
            row_scat_desc(m).start(add=True)
            cnt_scat_desc(m).start(add=True)

        def drain_scats(m):
            row_scat_desc(m).wait()
            cnt_scat_desc(m).wait()

        ei_desc(b0 + 0, 0).start()
        ei_desc(b0 + 1, 1).start()

        def stage(g, me, c, j, first_set):
            ei_desc(c, me).wait()

            @pl.when(g > 0)
            def _():
                drain_scats(first_set)
                drain_scats(first_set + 1)

            extract(me, first_set)

            @pl.when(j + 2 < nch)
            def _():
                ei_desc(c + 2, me).start()

            gather_desc(first_set).start()
            gather_desc(first_set + 1).start()

        def body(g, carry):
            ca = b0 + 2 * g
            stage(g, 0, ca, 2 * g, 0)
            stage(g, 1, ca + 1, 2 * g + 1, 2)
            for m in range(4):
                gather_desc(m).wait()
                fire_scats(m)
            return carry

        ngroups = nch // 2
        lax.fori_loop(0, ngroups, body, 0)

        @pl.when(nch % 2 == 1)
        def _():
            c = b0 + nch - 1
            ei_desc(c, 0).wait()
            drain_scats(0)
            drain_scats(1)
            extract(0, 0)
            gather_desc(0).start()
            gather_desc(1).start()
            for m in range(2):
                gather_desc(m).wait()
                fire_scats(m)
            drain_scats(0)
            drain_scats(1)

        @pl.when(nch % 2 == 0)
        def _():
            drain_scats(0)
            drain_scats(1)

        drain_scats(2)
        drain_scats(3)
        plsc.subcore_barrier()
        last = n - (ns - 1) * rps

        pltpu.sync_copy(
            cacc.at[pl.ds(sid * rps, rps)],
            outc_hbm.at[pl.ds(cid * np_ + sid * rps, rps)],
        )

        @pl.when(sid < ns - 1)
        def _():
            pltpu.sync_copy(
                acc.at[pl.ds(sid * rps, rps)],
                out_hbm.at[pl.ds(cid * n + sid * rps, rps)],
            )

        @pl.when(sid == ns - 1)
        def _():
            pltpu.sync_copy(
                acc.at[pl.ds((ns - 1) * rps, last)],
                out_hbm.at[pl.ds(cid * n + (ns - 1) * rps, last)],
            )

    return seg_sum(x, ei, zblk, zcnt)


_full_spec = lambda shape: pl.BlockSpec(shape, lambda i: (0,) * len(shape))


def _tc_pre(x, We, be2, Wr):
    n, d = x.shape
    h = We.shape[0]
    blk = 2000

    def body(x_r, we_r, be_r, wr_r, o_r):
        cdims = (((1,), (1,)), ((), ()))
        hb = lax.dot_general(x_r[...], we_r[...], cdims,
                             preferred_element_type=jnp.float32) + be_r[...]
        o_r[...] = lax.dot_general(hb, wr_r[...], cdims,
                                   preferred_element_type=jnp.float32)

    return pl.pallas_call(
        body,
        grid=(n // blk,),
        in_specs=[
            pl.BlockSpec((blk, d), lambda i: (i, 0)),
            _full_spec((h, d)),
            _full_spec((1, h)),
            _full_spec((h, h)),
        ],
        out_specs=pl.BlockSpec((blk, h), lambda i: (i, 0)),
        out_shape=jax.ShapeDtypeStruct((n, h), jnp.float32),
    )(x, We, be2, Wr)


def _tc_dense(hr, part, cnt2, We, be2, Wl, bl2, g2, b2, Wo, bo2):
    n, d = hr.shape
    h = We.shape[0]
    o = Wo.shape[0]
    blk = 2000
    grid = (n // blk,)
    nb = n // blk

    def body(hr_r, s0_r, s1_r, c_r, we_r, be_r, wl_r, bl_r, g_r, bt_r, wo_r, bo_r, o_r):
        cdims = (((1,), (1,)), ((), ()))
        s = s0_r[...] + s1_r[...]
        cnt = c_r[...]
        agg = lax.dot_general(s, we_r[...], cdims, preferred_element_type=jnp.float32) + cnt * be_r[...]
        mean = agg / jnp.maximum(cnt, 1.0)
        h2 = (
            lax.dot_general(mean, wl_r[...], cdims, preferred_element_type=jnp.float32)
            + bl_r[...]
            + hr_r[...]
        )
        mu = jnp.mean(h2, axis=-1, keepdims=True)
        zc = h2 - mu
        var = jnp.mean(zc * zc, axis=-1, keepdims=True)
        hn = zc * lax.rsqrt(var + 1e-5) * g_r[...] + bt_r[...]
        hn = jnp.maximum(hn, 0.0)
        o_r[...] = lax.dot_general(hn, wo_r[...], cdims, preferred_element_type=jnp.float32) + bo_r[...]

    return pl.pallas_call(
        body,
        grid=grid,
        in_specs=[
            pl.BlockSpec((blk, d), lambda i: (i, 0)),
            pl.BlockSpec((blk, d), lambda i: (i, 0)),
            pl.BlockSpec((blk, d), lambda i: (nb + i, 0)),
            pl.BlockSpec((blk, 1), lambda i: (i, 0)),
            _full_spec((h, d)),
            _full_spec((1, h)),
            _full_spec((h, h)),
            _full_spec((1, h)),
            _full_spec((1, h)),
            _full_spec((1, h)),
            _full_spec((o, h)),
            _full_spec((1, o)),
        ],
        out_specs=pl.BlockSpec((blk, o), lambda i: (i, 0)),
        out_shape=jax.ShapeDtypeStruct((n, o), jnp.float32),
    )(hr, part, part, cnt2, We, be2, Wl, bl2, g2, b2, Wo, bo2)


def kernel(x, edge_index, We, be, Wl, bl, Wr, gamma, beta, Wo, bo):
    n, d = x.shape
    ei = edge_index.astype(jnp.int32)
    rps = (-(-n // 16) + 15) // 16 * 16
    np_ = rps * 16
    zblk = jnp.zeros((rps, d), jnp.float32)
    zcnt = jnp.zeros((rps,), jnp.float32)
    hr = _tc_pre(x, We, be.reshape(1, -1), Wr)
    part, cnt = _sc_segment_sum(x, ei, zblk, zcnt)
    cnt_tot = cnt[:n] + cnt[np_ : np_ + n]
    return _tc_dense(
        hr, part, cnt_tot[:, None],
        We, be.reshape(1, -1), Wl, bl.reshape(1, -1),
        gamma.reshape(1, -1), beta.reshape(1, -1), Wo, bo.reshape(1, -1),
    )


# speedup vs baseline: 1.0106x; 1.0106x over previous
import functools

import jax
import jax.numpy as jnp
from jax import lax
from jax.experimental import pallas as pl
from jax.experimental.pallas import tpu as pltpu
from jax.experimental.pallas import tpu_sc as plsc

_CHUNK = 128


def _sc_segment_sum(x, ei, zblk, zcnt):
    n, d = x.shape
    nc2, e = ei.shape
    info = plsc.get_sparse_core_info()
    nc, ns = info.num_cores, info.num_subcores
    nw = nc * ns
    nch_all = e // _CHUNK
    q, r = divmod(nch_all, nw)
    half = _CHUNK // 2
    rps = zblk.shape[0]
    np_ = rps * ns
    assert nch_all * _CHUNK == e and nc2 == 2 and rps % 16 == 0 and np_ >= n
    assert 0 < n - (ns - 1) * rps <= rps and (n - (ns - 1) * rps) % 16 == 0
    assert n % 16 == 0 and q >= 4

    mesh = plsc.VectorSubcoreMesh(core_axis_name="c", subcore_axis_name="s")

    @functools.partial(
        pl.kernel,
        mesh=mesh,
        out_type=[
            jax.ShapeDtypeStruct((nc * n, d), jnp.float32),
            jax.ShapeDtypeStruct((nc * np_,), jnp.float32),
        ],
        scratch_types=[
            pltpu.VMEM_SHARED((np_, d), jnp.float32),
            pltpu.VMEM_SHARED((np_,), jnp.float32),
        ]
        + [pltpu.VMEM((2, _CHUNK), jnp.int32)] * 2
        + [pltpu.VMEM((half,), jnp.int32)] * 8
        + [pltpu.VMEM((half, d), jnp.float32)] * 4
        + [pltpu.VMEM((half,), jnp.float32)]
        + [pltpu.SemaphoreType.DMA] * 10,
    )
    def seg_sum(x_hbm, ei_hbm, z_hbm, zc_hbm, out_hbm, outc_hbm,
                acc, cacc, eib0, eib1, si0, si1, si2, si3, di0, di1, di2, di3,
                rows0, rows1, rows2, rows3, ones_v,
                sem_e0, sem_e1, sg0, sg1, sg2, sg3, ss0, ss1, ss2, ss3):
        eib = (eib0, eib1)
        sidx = (si0, si1, si2, si3)
        didx = (di0, di1, di2, di3)
        rows = (rows0, rows1, rows2, rows3)
        sem_e = (sem_e0, sem_e1)
        sem_g = (sg0, sg1, sg2, sg3)
        sem_s = (ss0, ss1, ss2, ss3)
        cid = lax.axis_index("c")
        sid = lax.axis_index("s")
        pltpu.sync_copy(z_hbm, acc.at[pl.ds(sid * rps, rps)])
        pltpu.sync_copy(zc_hbm, cacc.at[pl.ds(sid * rps, rps)])
        for j in range(half // 16):
            ones_v[pl.ds(16 * j, 16)] = jnp.ones((16,), jnp.float32)
        wid = cid * ns + sid
        nch = q + jnp.where(wid < r, 1, 0)
        b0 = wid * q + jnp.minimum(wid, r)
        plsc.subcore_barrier()

        def ei_desc(c, me):
            return pltpu.make_async_copy(
                ei_hbm.at[:, pl.ds(c * _CHUNK, _CHUNK)], eib[me], sem_e[me])

        def extract(me, m):
            for hh in range(2):
                for j in range(half // 16):
                    sl = pl.ds(hh * half + 16 * j, 16)
                    so = pl.ds(16 * j, 16)
                    sidx[m + hh][so] = eib[me][0, sl]
                    didx[m + hh][so] = eib[me][1, sl]

        def gather_desc(m):
            return pltpu.make_async_copy(x_hbm.at[sidx[m]], rows[m], sem_g[m])

        def row_scat_desc(m):
            return pltpu.make_async_copy(rows[m], acc.at[didx[m]], sem_s[m])

        def cnt_scat_desc(m):
            return pltpu.make_async_copy(ones_v, cacc.at[didx[m]], sem_s[m])

        def fire_scats(m):
            row_scat_desc(m).start(add=True)
            cnt_scat_desc(m).start(add=True)

        def drain_scats(m):
            row_scat_desc(m).wait()
            cnt_scat_desc(m).wait()

        ei_desc(b0 + 0, 0).start()
        ei_desc(b0 + 1, 1).start()

        def stage(g, me, c, j, first_set):
            ei_desc(c, me).wait()

            @pl.when(g > 0)
            def _():
                drain_scats(first_set)
                drain_scats(first_set + 1)

            extract(me, first_set)

            @pl.when(j + 2 < nch)
            def _():
                ei_desc(c + 2, me).start()

            gather_desc(first_set).start()
            gather_desc(first_set + 1).start()

        def body(g, carry):
            ca = b0 + 2 * g
            stage(g, 0, ca, 2 * g, 0)
            stage(g, 1, ca + 1, 2 * g + 1, 2)
            for m in range(4):
                gather_desc(m).wait()
                fire_scats(m)
            return carry

        ngroups = nch // 2
        lax.fori_loop(0, ngroups, body, 0)

        @pl.when(nch % 2 == 1)
        def _():
            c = b0 + nch - 1
            ei_desc(c, 0).wait()
            drain_scats(0)
            drain_scats(1)
            extract(0, 0)
            gather_desc(0).start()
            gather_desc(1).start()
            for m in range(2):
                gather_desc(m).wait()
                fire_scats(m)
            drain_scats(0)
            drain_scats(1)

        @pl.when(nch % 2 == 0)
        def _():
            drain_scats(0)
            drain_scats(1)

        drain_scats(2)
        drain_scats(3)
        plsc.subcore_barrier()
        last = n - (ns - 1) * rps

        pltpu.sync_copy(
            cacc.at[pl.ds(sid * rps, rps)],
            outc_hbm.at[pl.ds(cid * np_ + sid * rps, rps)],
        )

        @pl.when(sid < ns - 1)
        def _():
            pltpu.sync_copy(
                acc.at[pl.ds(sid * rps, rps)],
                out_hbm.at[pl.ds(cid * n + sid * rps, rps)],
            )

        @pl.when(sid == ns - 1)
        def _():
            pltpu.sync_copy(
                acc.at[pl.ds((ns - 1) * rps, last)],
                out_hbm.at[pl.ds(cid * n + (ns - 1) * rps, last)],
            )

    return seg_sum(x, ei, zblk, zcnt)


def _tc_dense(x, part, cnt2, We, be2, Wl, bl2, Wr, g2, b2, Wo, bo2):
    n, d = x.shape
    h = We.shape[0]
    o = Wo.shape[0]
    blk = 2000
    grid = (n // blk,)
    nb = n // blk

    def body(x_r, s0_r, s1_r, c_r, we_r, be_r, wl_r, bl_r, wr_r, g_r, bt_r, wo_r, bo_r, o_r):
        cdims = (((1,), (1,)), ((), ()))
        xb = x_r[...]
        s = s0_r[...] + s1_r[...]
        cnt = c_r[...]
        hb = lax.dot_general(xb, we_r[...], cdims, preferred_element_type=jnp.float32) + be_r[...]
        agg = lax.dot_general(s, we_r[...], cdims, preferred_element_type=jnp.float32) + cnt * be_r[...]
        mean = agg / jnp.maximum(cnt, 1.0)
        h2 = (
            lax.dot_general(mean, wl_r[...], cdims, preferred_element_type=jnp.float32)
            + bl_r[...]
            + lax.dot_general(hb, wr_r[...], cdims, preferred_element_type=jnp.float32)
        )
        mu = jnp.mean(h2, axis=-1, keepdims=True)
        zc = h2 - mu
        var = jnp.mean(zc * zc, axis=-1, keepdims=True)
        hn = zc * lax.rsqrt(var + 1e-5) * g_r[...] + bt_r[...]
        hn = jnp.maximum(hn, 0.0)
        o_r[...] = lax.dot_general(hn, wo_r[...], cdims, preferred_element_type=jnp.float32) + bo_r[...]

    full = lambda shape: pl.BlockSpec(shape, lambda i: (0,) * len(shape))
    return pl.pallas_call(
        body,
        grid=grid,
        in_specs=[
            pl.BlockSpec((blk, d), lambda i: (i, 0)),
            pl.BlockSpec((blk, d), lambda i: (i, 0)),
            pl.BlockSpec((blk, d), lambda i: (nb + i, 0)),
            pl.BlockSpec((blk, 1), lambda i: (i, 0)),
            full((h, d)),
            full((1, h)),
            full((h, h)),
            full((1, h)),
            full((h, h)),
            full((1, h)),
            full((1, h)),
            full((o, h)),
            full((1, o)),
        ],
        out_specs=pl.BlockSpec((blk, o), lambda i: (i, 0)),
        out_shape=jax.ShapeDtypeStruct((n, o), jnp.float32),
    )(x, part, part, cnt2, We, be2, Wl, bl2, Wr, g2, b2, Wo, bo2)


def kernel(x, edge_index, We, be, Wl, bl, Wr, gamma, beta, Wo, bo):
    n, d = x.shape
    ei = edge_index.astype(jnp.int32)
    rps = (-(-n // 16) + 15) // 16 * 16
    np_ = rps * 16
    zblk = jnp.zeros((rps, d), jnp.float32)
    zcnt = jnp.zeros((rps,), jnp.float32)
    part, cnt = _sc_segment_sum(x, ei, zblk, zcnt)
    cnt_tot = cnt[:n] + cnt[np_ : np_ + n]
    return _tc_dense(
        x, part, cnt_tot[:, None],
        We, be.reshape(1, -1), Wl, bl.reshape(1, -1), Wr,
        gamma.reshape(1, -1), beta.reshape(1, -1), Wo, bo.reshape(1, -1),
    )
